# baseline (speedup 1.0000x reference)
import functools

import jax
import jax.numpy as jnp
from jax import lax
from jax.experimental import pallas as pl
from jax.experimental.pallas import tpu as pltpu
from jax.experimental.pallas import tpu_sc as plsc

N_TARGETS = 4096
N_FEATURES = 65536
NNZ = 4194304
BETA = 2.0 / 3.0
GAMMA = -0.1
ZETA = 1.1

NC = 2
NS = 16
NW = NC * NS
L = 16

NNZ_PER_TILE = NNZ // NW
CHUNK = 8192
N_CHUNKS = NNZ_PER_TILE // CHUNK
NBUF = 2


def _weights_body(log_weight_ref, log_alpha_ref, w_ref):
    s = jax.nn.sigmoid(log_alpha_ref[...] / BETA)
    gates = jnp.clip(s * (ZETA - GAMMA) + GAMMA, 0.0, 1.0)
    w_ref[...] = jnp.exp(log_weight_ref[...]) * gates


def _compute_weights(log_weight, log_alpha):
    lw = log_weight.reshape(512, 128)
    la = log_alpha.reshape(512, 128)
    w = pl.pallas_call(
        _weights_body,
        out_shape=jax.ShapeDtypeStruct((512, 128), jnp.float32),
    )(lw, la)
    return w.reshape(N_FEATURES)


def _lane_gather(v, idx):
    dnums = lax.GatherDimensionNumbers(
        offset_dims=(), collapsed_slice_dims=(0,), start_index_map=(0,))
    return lax.gather(v, idx[:, None], dnums, (1,),
                      mode=lax.GatherScatterMode.PROMISE_IN_BOUNDS)


def _sc_body(cols_hbm, vals_hbm, rows_hbm, w_hbm, out_hbm,
             table_v, cols_v, vals_v, rows_v, acc_v, sems):
    wid = lax.axis_index("s") * NC + lax.axis_index("c")
    base = wid * NNZ_PER_TILE

    def start_chunk(k, b):
        off = base + k * CHUNK
        pltpu.async_copy(cols_hbm.at[pl.ds(off, CHUNK)], cols_v.at[b], sems.at[b])
        pltpu.async_copy(vals_hbm.at[pl.ds(off, CHUNK)], vals_v.at[b], sems.at[b])
        pltpu.async_copy(rows_hbm.at[pl.ds(off, CHUNK)], rows_v.at[b], sems.at[b])

    def wait_chunk(b):
        pltpu.make_async_copy(cols_hbm.at[pl.ds(0, CHUNK)], cols_v.at[b], sems.at[b]).wait()
        pltpu.make_async_copy(vals_hbm.at[pl.ds(0, CHUNK)], vals_v.at[b], sems.at[b]).wait()
        pltpu.make_async_copy(rows_hbm.at[pl.ds(0, CHUNK)], rows_v.at[b], sems.at[b]).wait()

    start_chunk(0, 0)
    pltpu.sync_copy(w_hbm, table_v)

    zeros = jnp.zeros((L,), jnp.float32)
    for i in range(N_TARGETS // L):
        acc_v[pl.ds(i * L, L)] = zeros

    for k in range(N_CHUNKS):
        b = k % NBUF
        if k + 1 < N_CHUNKS:
            start_chunk(k + 1, (k + 1) % NBUF)
        wait_chunk(b)

        lanes = lax.iota(jnp.int32, L)
        last_lane = lanes == (L - 1)
        next_idx = jnp.minimum(lanes + 1, L - 1)

        def group_body(g, b=b):
            idx = cols_v[b, pl.ds(g * L, L)]
            w = plsc.load_gather(table_v, [idx])
            p = vals_v[b, pl.ds(g * L, L)] * w
            r = rows_v[b, pl.ds(g * L, L)]
            r_next = _lane_gather(r, next_idx)
            d = r != r_next
            last = d | last_lane
            c = plsc.cumsum(p)
            plsc.addupdate_scatter(acc_v, [r], c, mask=last)
            plsc.addupdate_scatter(acc_v, [r_next], -c, mask=d)

        plsc.parallel_loop(0, CHUNK // L, unroll=12)(group_body)

    pltpu.sync_copy(acc_v, out_hbm.at[wid])


def _sc_spmv(cols, vals, rows, w):
    mesh = plsc.VectorSubcoreMesh(core_axis_name="c", subcore_axis_name="s")
    kern = pl.kernel(
        _sc_body,
        out_type=jax.ShapeDtypeStruct((NW, N_TARGETS), jnp.float32),
        mesh=mesh,
        scratch_types=[
            pltpu.VMEM((N_FEATURES,), jnp.float32),
            pltpu.VMEM((NBUF, CHUNK), jnp.int32),
            pltpu.VMEM((NBUF, CHUNK), jnp.float32),
            pltpu.VMEM((NBUF, CHUNK), jnp.int32),
            pltpu.VMEM((N_TARGETS,), jnp.float32),
            pltpu.SemaphoreType.DMA((NBUF,)),
        ],
        compiler_params=pltpu.CompilerParams(needs_layout_passes=False),
    )
    return kern(cols, vals, rows, w)


def _reduce_body(part_ref, y_ref):
    y_ref[...] = jnp.sum(part_ref[...], axis=0, keepdims=True)


def _reduce_partials(partials):
    y = pl.pallas_call(
        _reduce_body,
        out_shape=jax.ShapeDtypeStruct((1, N_TARGETS), jnp.float32),
    )(partials)
    return y.reshape(N_TARGETS)


@jax.jit
def kernel(M_rows, M_cols, M_vals, log_weight, log_alpha):
    w = _compute_weights(log_weight, log_alpha)
    partials = _sc_spmv(M_cols, M_vals, M_rows, w)
    return _reduce_partials(partials)


# speedup vs baseline: 1.0727x; 1.0727x over previous
import functools

import jax
import jax.numpy as jnp
from jax import lax
from jax.experimental import pallas as pl
from jax.experimental.pallas import tpu as pltpu
from jax.experimental.pallas import tpu_sc as plsc

N_TARGETS = 4096
N_FEATURES = 65536
NNZ = 4194304
BETA = 2.0 / 3.0
GAMMA = -0.1
ZETA = 1.1

NC = 2
NS = 16
NW = NC * NS
L = 16

NNZ_PER_TILE = NNZ // NW
CHUNK = 8192
N_CHUNKS = NNZ_PER_TILE // CHUNK
NBUF = 2


def _weights_body(log_weight_ref, log_alpha_ref, w_ref):
    s = jax.nn.sigmoid(log_alpha_ref[...] / BETA)
    gates = jnp.clip(s * (ZETA - GAMMA) + GAMMA, 0.0, 1.0)
    w_ref[...] = jnp.exp(log_weight_ref[...]) * gates


def _compute_weights(log_weight, log_alpha):
    lw = log_weight.reshape(512, 128)
    la = log_alpha.reshape(512, 128)
    w = pl.pallas_call(
        _weights_body,
        out_shape=jax.ShapeDtypeStruct((512, 128), jnp.float32),
    )(lw, la)
    return w.reshape(N_FEATURES)


def _lane_gather(v, idx):
    dnums = lax.GatherDimensionNumbers(
        offset_dims=(), collapsed_slice_dims=(0,), start_index_map=(0,))
    return lax.gather(v, idx[:, None], dnums, (1,),
                      mode=lax.GatherScatterMode.PROMISE_IN_BOUNDS)


def _sc_body(cols_hbm, vals_hbm, rows_hbm, w_hbm, out_hbm,
             table_v, cols_v, vals_v, rows_v, acc_v, sems):
    wid = lax.axis_index("s") * NC + lax.axis_index("c")
    base = wid * NNZ_PER_TILE

    def start_chunk(k, b):
        off = base + k * CHUNK
        pltpu.async_copy(cols_hbm.at[pl.ds(off, CHUNK)], cols_v.at[b], sems.at[b])
        pltpu.async_copy(vals_hbm.at[pl.ds(off, CHUNK)], vals_v.at[b], sems.at[b])
        pltpu.async_copy(rows_hbm.at[pl.ds(off, CHUNK)], rows_v.at[b], sems.at[b])

    def wait_chunk(b):
        pltpu.make_async_copy(cols_hbm.at[pl.ds(0, CHUNK)], cols_v.at[b], sems.at[b]).wait()
        pltpu.make_async_copy(vals_hbm.at[pl.ds(0, CHUNK)], vals_v.at[b], sems.at[b]).wait()
        pltpu.make_async_copy(rows_hbm.at[pl.ds(0, CHUNK)], rows_v.at[b], sems.at[b]).wait()

    start_chunk(0, 0)
    pltpu.sync_copy(w_hbm, table_v)

    zeros = jnp.zeros((L,), jnp.float32)
    for i in range(N_TARGETS // L):
        acc_v[pl.ds(i * L, L)] = zeros

    for k in range(N_CHUNKS):
        b = k % NBUF
        if k + 1 < N_CHUNKS:
            start_chunk(k + 1, (k + 1) % NBUF)
        wait_chunk(b)

        lanes = lax.iota(jnp.int32, L)
        last_lane = lanes == (L - 1)
        next_idx = jnp.minimum(lanes + 1, L - 1)

        def group_body(g, b=b):
            idx = cols_v[b, pl.ds(g * L, L)]
            w = plsc.load_gather(table_v, [idx])
            p = vals_v[b, pl.ds(g * L, L)] * w
            r = rows_v[b, pl.ds(g * L, L)]
            r_next = _lane_gather(r, next_idx)
            d = r != r_next
            last = d | last_lane
            c = plsc.cumsum(p)
            plsc.addupdate_scatter(acc_v, [r], c, mask=last)
            plsc.addupdate_scatter(acc_v, [r_next], -c, mask=d)

        plsc.parallel_loop(0, CHUNK // L, unroll=6)(group_body)

    pltpu.sync_copy(acc_v, out_hbm.at[wid])


def _sc_spmv(cols, vals, rows, w):
    mesh = plsc.VectorSubcoreMesh(core_axis_name="c", subcore_axis_name="s")
    kern = pl.kernel(
        _sc_body,
        out_type=jax.ShapeDtypeStruct((NW, N_TARGETS), jnp.float32),
        mesh=mesh,
        scratch_types=[
            pltpu.VMEM((N_FEATURES,), jnp.float32),
            pltpu.VMEM((NBUF, CHUNK), jnp.int32),
            pltpu.VMEM((NBUF, CHUNK), jnp.float32),
            pltpu.VMEM((NBUF, CHUNK), jnp.int32),
            pltpu.VMEM((N_TARGETS,), jnp.float32),
            pltpu.SemaphoreType.DMA((NBUF,)),
        ],
        compiler_params=pltpu.CompilerParams(needs_layout_passes=False),
    )
    return kern(cols, vals, rows, w)


def _reduce_body(part_ref, y_ref):
    y_ref[...] = jnp.sum(part_ref[...], axis=0, keepdims=True)


def _reduce_partials(partials):
    y = pl.pallas_call(
        _reduce_body,
        out_shape=jax.ShapeDtypeStruct((1, N_TARGETS), jnp.float32),
    )(partials)
    return y.reshape(N_TARGETS)


@jax.jit
def kernel(M_rows, M_cols, M_vals, log_weight, log_alpha):
    w = _compute_weights(log_weight, log_alpha)
    partials = _sc_spmv(M_cols, M_vals, M_rows, w)
    return _reduce_partials(partials)
